# baseline (speedup 1.0000x reference)
import functools

import jax
import jax.numpy as jnp
from jax import lax
from jax.experimental import pallas as pl
from jax.experimental.pallas import tpu as pltpu
from jax.experimental.pallas import tpu_sc as plsc

EPS = 1e-5

_NC = 2
_NS = 16
_NW = _NC * _NS


def _sc_gather(table, flat_ids):
    n, d = flat_ids.shape[0], table.shape[1]
    b_per_w = n // _NW
    chunk = 64
    n_chunks = b_per_w // chunk
    mesh = plsc.VectorSubcoreMesh(core_axis_name="c", subcore_axis_name="s")

    @functools.partial(
        pl.kernel,
        mesh=mesh,
        out_type=jax.ShapeDtypeStruct((n, d), jnp.float32),
        scratch_types=[
            pltpu.VMEM((b_per_w,), jnp.int32),
            pltpu.VMEM((chunk, d), jnp.float32),
            pltpu.VMEM((chunk, d), jnp.float32),
            pltpu.SemaphoreType.DMA,
            pltpu.SemaphoreType.DMA,
            pltpu.SemaphoreType.DMA,
            pltpu.SemaphoreType.DMA,
        ],
    )
    def gather_kernel(table_hbm, idx_hbm, out_hbm, idx_v, rows0, rows1,
                      g0, g1, w0, w1):
        wid = lax.axis_index("s") * _NC + lax.axis_index("c")
        base = wid * b_per_w
        bufs, gsems, wsems = [rows0, rows1], [g0, g1], [w0, w1]
        pltpu.sync_copy(idx_hbm.at[pl.ds(base, b_per_w)], idx_v)

        def gather_start(c):
            idx_c = idx_v.at[pl.ds(c * chunk, chunk)]
            return pltpu.async_copy(table_hbm.at[idx_c], bufs[c % 2],
                                    gsems[c % 2])

        def write_start(c):
            return pltpu.async_copy(bufs[c % 2],
                                    out_hbm.at[pl.ds(base + c * chunk, chunk)],
                                    wsems[c % 2])

        gathers = [gather_start(0)]
        writes = [None, None]
        for c in range(n_chunks):
            gathers[c].wait()
            if c >= 1:
                writes[(c - 1) % 2].wait()
            if c + 1 < n_chunks:
                gathers.append(gather_start(c + 1))
            writes[c % 2] = write_start(c)
        writes[(n_chunks - 1) % 2].wait()

    return gather_kernel(table, flat_ids)


def _ln_body(w_ref, t_ref, pos_ref, ttab_ref, sc_ref, of_ref, out_ref):
    x = w_ref[0] + pos_ref[...]
    t = t_ref[0, 0, :]
    mask = t[:, None] == 0
    x = x + jnp.where(mask, ttab_ref[0:1, :], ttab_ref[1:2, :])
    mean = jnp.mean(x, axis=1, keepdims=True)
    xc = x - mean
    var = jnp.mean(xc * xc, axis=1, keepdims=True)
    y = xc * lax.rsqrt(var + EPS)
    out_ref[0] = y * sc_ref[...] + of_ref[...]


def _tc_add_ln(word_emb, token_type_ids, pos_table, type_table, ln_scale, ln_offset):
    b, s, d = word_emb.shape
    tt3 = token_type_ids.reshape(b, 1, s)
    return pl.pallas_call(
        _ln_body,
        grid=(b,),
        in_specs=[
            pl.BlockSpec((1, s, d), lambda i: (i, 0, 0)),
            pl.BlockSpec((1, 1, s), lambda i: (i, 0, 0)),
            pl.BlockSpec((s, d), lambda i: (0, 0)),
            pl.BlockSpec((2, d), lambda i: (0, 0)),
            pl.BlockSpec((1, d), lambda i: (0, 0)),
            pl.BlockSpec((1, d), lambda i: (0, 0)),
        ],
        out_specs=pl.BlockSpec((1, s, d), lambda i: (i, 0, 0)),
        out_shape=jax.ShapeDtypeStruct((b, s, d), jnp.float32),
    )(word_emb, tt3, pos_table, type_table,
      ln_scale.reshape(1, d), ln_offset.reshape(1, d))


@jax.jit
def kernel(input_ids, token_type_ids, word_table, pos_table, type_table, ln_scale, ln_offset):
    b, s = input_ids.shape
    d = word_table.shape[1]
    flat_ids = input_ids.reshape(b * s)
    word_emb = _sc_gather(word_table, flat_ids).reshape(b, s, d)
    return _tc_add_ln(word_emb, token_type_ids, pos_table[:s], type_table,
                      ln_scale, ln_offset)


# speedup vs baseline: 1.0926x; 1.0926x over previous
import functools

import jax
import jax.numpy as jnp
from jax import lax
from jax.experimental import pallas as pl
from jax.experimental.pallas import tpu as pltpu
from jax.experimental.pallas import tpu_sc as plsc

EPS = 1e-5

_NC = 2
_NS = 16
_NW = _NC * _NS


def _sc_gather(table, flat_ids):
    n, d = flat_ids.shape[0], table.shape[1]
    b_per_w = n // _NW
    chunk = 64
    n_chunks = b_per_w // chunk
    mesh = plsc.VectorSubcoreMesh(core_axis_name="c", subcore_axis_name="s")

    @functools.partial(
        pl.kernel,
        mesh=mesh,
        out_type=jax.ShapeDtypeStruct((n, d), jnp.float32),
        scratch_types=[
            pltpu.VMEM((b_per_w,), jnp.int32),
            pltpu.VMEM((chunk, d), jnp.float32),
            pltpu.VMEM((chunk, d), jnp.float32),
            pltpu.SemaphoreType.DMA,
            pltpu.SemaphoreType.DMA,
            pltpu.SemaphoreType.DMA,
            pltpu.SemaphoreType.DMA,
        ],
    )
    def gather_kernel(table_hbm, idx_hbm, out_hbm, idx_v, rows0, rows1,
                      g0, g1, w0, w1):
        wid = lax.axis_index("s") * _NC + lax.axis_index("c")
        base = wid * b_per_w
        bufs, gsems, wsems = [rows0, rows1], [g0, g1], [w0, w1]
        pltpu.sync_copy(idx_hbm.at[pl.ds(base, b_per_w)], idx_v)

        def gather_start(c):
            idx_c = idx_v.at[pl.ds(c * chunk, chunk)]
            return pltpu.async_copy(table_hbm.at[idx_c], bufs[c % 2],
                                    gsems[c % 2])

        def write_start(c):
            return pltpu.async_copy(bufs[c % 2],
                                    out_hbm.at[pl.ds(base + c * chunk, chunk)],
                                    wsems[c % 2])

        gathers = [gather_start(0)]
        writes = [None, None]
        for c in range(n_chunks):
            gathers[c].wait()
            if c >= 1:
                writes[(c - 1) % 2].wait()
            if c + 1 < n_chunks:
                gathers.append(gather_start(c + 1))
            writes[c % 2] = write_start(c)
        writes[(n_chunks - 1) % 2].wait()

    return gather_kernel(table, flat_ids)


def _ln_body(w_ref, t_ref, pos_ref, ttab_ref, sc_ref, of_ref, out_ref):
    x = w_ref[...] + pos_ref[...]
    mask = t_ref[...] == 0
    x = x + jnp.where(mask, ttab_ref[:, 0:1, :], ttab_ref[:, 1:2, :])
    mean = jnp.mean(x, axis=2, keepdims=True)
    xc = x - mean
    var = jnp.mean(xc * xc, axis=2, keepdims=True)
    y = xc * lax.rsqrt(var + EPS)
    out_ref[...] = y * sc_ref[...] + of_ref[...]


def _tc_add_ln(word_emb, token_type_ids, pos_table, type_table, ln_scale, ln_offset):
    b, s, d = word_emb.shape
    bb = 4
    tt3 = token_type_ids.reshape(b, s, 1)
    return pl.pallas_call(
        _ln_body,
        grid=(b // bb,),
        in_specs=[
            pl.BlockSpec((bb, s, d), lambda i: (i, 0, 0)),
            pl.BlockSpec((bb, s, 1), lambda i: (i, 0, 0)),
            pl.BlockSpec((1, s, d), lambda i: (0, 0, 0)),
            pl.BlockSpec((1, 2, d), lambda i: (0, 0, 0)),
            pl.BlockSpec((1, 1, d), lambda i: (0, 0, 0)),
            pl.BlockSpec((1, 1, d), lambda i: (0, 0, 0)),
        ],
        out_specs=pl.BlockSpec((bb, s, d), lambda i: (i, 0, 0)),
        out_shape=jax.ShapeDtypeStruct((b, s, d), jnp.float32),
    )(word_emb, tt3, pos_table.reshape(1, s, d), type_table.reshape(1, 2, d),
      ln_scale.reshape(1, 1, d), ln_offset.reshape(1, 1, d))


@jax.jit
def kernel(input_ids, token_type_ids, word_table, pos_table, type_table, ln_scale, ln_offset):
    b, s = input_ids.shape
    d = word_table.shape[1]
    flat_ids = input_ids.reshape(b * s)
    word_emb = _sc_gather(word_table, flat_ids).reshape(b, s, d)
    return _tc_add_ln(word_emb, token_type_ids, pos_table[:s], type_table,
                      ln_scale, ln_offset)
